# baseline (speedup 1.0000x reference)
import jax
import jax.numpy as jnp
from jax import lax
from jax.experimental import pallas as pl
from jax.experimental.pallas import tpu as pltpu
from jax.experimental.pallas import tpu_sc as plsc

N = 100000
HIDDEN = 128
VOCAB = 64
CHUNK = 128
NC = 2
NS = 16
NW = NC * NS
LANES = 16
CPW = 25
RPW = CPW * CHUNK
NPAD = NW * RPW
NBUF = 3


def _pair_body(w0, w1, w2, w3, t01, t23):
    t01[...] = w0[...][:, None, :] + w1[...][None, :, :]
    t23[...] = w2[...][:, None, :] + w3[...][None, :, :]


def _build_pair_tables(W0, W1, W2, W3):
    t01, t23 = pl.pallas_call(
        _pair_body,
        out_shape=[
            jax.ShapeDtypeStruct((VOCAB, VOCAB, HIDDEN), jnp.float32),
            jax.ShapeDtypeStruct((VOCAB, VOCAB, HIDDEN), jnp.float32),
        ],
    )(W0, W1, W2, W3)
    return (t01.reshape(VOCAB * VOCAB, HIDDEN),
            t23.reshape(VOCAB * VOCAB, HIDDEN))


def _sc_body(x0, x1, x2, x3, t01, t23, out,
             xa, xb, i01, i23,
             b01_0, b01_1, b01_2, b23_0, b23_1, b23_2,
             gs0, gs1, gs2, ws0, ws1, ws2):
    b01s = (b01_0, b01_1, b01_2)
    b23s = (b23_0, b23_1, b23_2)
    gsem = (gs0, gs1, gs2)
    wsem = (ws0, ws1, ws2)

    wid = lax.axis_index("s") * NC + lax.axis_index("c")
    base = pl.multiple_of(wid * RPW, RPW)

    pltpu.sync_copy(x0.at[pl.ds(base, RPW)], xa)
    pltpu.sync_copy(x1.at[pl.ds(base, RPW)], xb)

    def fold(j, c):
        s = pl.ds(pl.multiple_of(j * LANES, LANES), LANES)
        i01[s] = xa[s] * VOCAB + xb[s]
        return c

    lax.fori_loop(0, RPW // LANES, fold, 0)
    pltpu.sync_copy(x2.at[pl.ds(base, RPW)], xa)
    pltpu.sync_copy(x3.at[pl.ds(base, RPW)], xb)

    def fold2(j, c):
        s = pl.ds(pl.multiple_of(j * LANES, LANES), LANES)
        i23[s] = xa[s] * VOCAB + xb[s]
        return c

    lax.fori_loop(0, RPW // LANES, fold2, 0)

    gcp = {}

    def fire_gathers(k):
        p = k % NBUF
        s = pl.ds(k * CHUNK, CHUNK)
        gcp[k] = (pltpu.async_copy(t01.at[i01.at[s]], b01s[p], gsem[p]),
                  pltpu.async_copy(t23.at[i23.at[s]], b23s[p], gsem[p]))

    wcp = [None] * NBUF
    fire_gathers(0)
    fire_gathers(1)

    for k in range(CPW):
        pf = (k + 2) % NBUF
        if k + 2 < CPW:
            if wcp[pf] is not None:
                wcp[pf].wait()
            fire_gathers(k + 2)
        p = k % NBUF
        c01, c23 = gcp.pop(k)
        c01.wait()
        c23.wait()
        b01, b23 = b01s[p], b23s[p]

        def add_row(r, c2, b01=b01, b23=b23):
            for j in range(HIDDEN // LANES):
                s = pl.ds(j * LANES, LANES)
                b01[r, s] = b01[r, s] + b23[r, s]
            return c2

        lax.fori_loop(0, CHUNK, add_row, 0)
        wcp[p] = pltpu.async_copy(
            b01, out.at[pl.ds(base + k * CHUNK, CHUNK)], wsem[p])

    for p in range(NBUF):
        if wcp[p] is not None:
            wcp[p].wait()


def kernel(x, W0, W1, W2, W3):
    xT = jnp.pad(x.astype(jnp.int32).T, ((0, 0), (0, NPAD - N)))
    x0, x1, x2, x3 = xT[0], xT[1], xT[2], xT[3]
    t01, t23 = _build_pair_tables(W0, W1, W2, W3)
    mesh = plsc.VectorSubcoreMesh(core_axis_name="c", subcore_axis_name="s")
    f = pl.kernel(
        _sc_body,
        mesh=mesh,
        out_type=jax.ShapeDtypeStruct((NPAD, HIDDEN), jnp.float32),
        scratch_types=[
            pltpu.VMEM((RPW,), jnp.int32),
            pltpu.VMEM((RPW,), jnp.int32),
            pltpu.VMEM((RPW,), jnp.int32),
            pltpu.VMEM((RPW,), jnp.int32),
            pltpu.VMEM((CHUNK, HIDDEN), jnp.float32),
            pltpu.VMEM((CHUNK, HIDDEN), jnp.float32),
            pltpu.VMEM((CHUNK, HIDDEN), jnp.float32),
            pltpu.VMEM((CHUNK, HIDDEN), jnp.float32),
            pltpu.VMEM((CHUNK, HIDDEN), jnp.float32),
            pltpu.VMEM((CHUNK, HIDDEN), jnp.float32),
            pltpu.SemaphoreType.DMA,
            pltpu.SemaphoreType.DMA,
            pltpu.SemaphoreType.DMA,
            pltpu.SemaphoreType.DMA,
            pltpu.SemaphoreType.DMA,
            pltpu.SemaphoreType.DMA,
        ],
    )
    outp = f(x0, x1, x2, x3, t01, t23)
    return outp[:N]
